# baseline (speedup 1.0000x reference)
import numpy as np
import jax
import jax.numpy as jnp
from jax.experimental import pallas as pl
from jax.experimental.pallas import tpu as pltpu

B = 1024
N = 11
T = 10
IN_DIM = 4
D = 256
ZDIM = 32
BT = 32
R = BT * N
GRID = B // BT

_HP = jax.lax.Precision.HIGHEST


def _pe_const():
    position = np.arange(200, dtype=np.float32)[:, None]
    div_term = np.exp(np.arange(0, D, 2, dtype=np.float32) * (-np.log(10000.0) / D))
    pe = np.zeros((200, D), dtype=np.float32)
    pe[:, 0::2] = np.sin(position * div_term)
    pe[:, 1::2] = np.cos(position * div_term)
    return jnp.asarray(pe[:T])


def _cat3_const():
    cat3 = np.zeros((N, 3), dtype=np.float32)
    cat3[0:5, 0] = 1.0
    cat3[5:10, 1] = 1.0
    cat3[10, 2] = 1.0
    return jnp.asarray(cat3)


def _relu(x):
    return jnp.maximum(x, 0.0)


def _bfdot(a, b):
    return jax.lax.dot_general(a.astype(jnp.bfloat16), b, (((1,), (0,)), ((), ())),
                               preferred_element_type=jnp.float32)


def _block_kernel(x_ref, pf_ref,
                  wf_ref, bf_ref, wpet_ref, peb_ref, w2_ref, b2_ref,
                  w3a_ref, catb_ref,
                  wo1a_ref, wo1b_ref, bo1_ref, wo2_ref, bo2_ref,
                  wh1a_ref, bh1a_ref, wh1b_ref, bh1b_ref,
                  wh2a_ref, bh2a_ref, wh2b_ref, bh2b_ref,
                  wout_ref, bout_ref, wqz_ref, bqz_ref,
                  qz_ref, h_out_ref):
    x = x_ref[:]
    tf = _bfdot(x, wf_ref[:]) + bf_ref[:]
    tfp = _bfdot(tf, wpet_ref[:])
    f0 = b2_ref[:]
    for t in range(T):
        f0 = f0 + _bfdot(tfp[t * R:(t + 1) * R, :] + peb_ref[t:t + 1, :],
                         w2_ref[t * D:(t + 1) * D, :])
    ftraj = _bfdot(f0, w3a_ref[:]) + catb_ref[:]

    nrm = jnp.sqrt(jnp.sum(ftraj * ftraj, axis=1, keepdims=True))
    q = (ftraj / jnp.maximum(nrm, 1e-12)).astype(jnp.bfloat16)
    corr = jax.lax.dot_general(q, q, (((1,), (1,)), ((), ())),
                               preferred_element_type=jnp.float32)

    rs = jax.lax.broadcasted_iota(jnp.int32, (R, R), 0) // N
    cs = jax.lax.broadcasted_iota(jnp.int32, (R, R), 1) // N
    mask = rs == cs
    maskf = mask.astype(jnp.float32)

    mean_rows = jnp.dot(maskf, ftraj) * (1.0 / N)

    m = _relu(jnp.dot(ftraj, wo1a_ref[:])
              + jnp.dot(mean_rows, wo1b_ref[:]) + bo1_ref[:])
    inter = _relu(jnp.dot(m, wo2_ref[:]) + bo2_ref[:])

    colidx = jax.lax.broadcasted_iota(jnp.int32, (R, R), 1)
    nidx = jax.lax.broadcasted_iota(jnp.int32, (R, N), 1)
    rowscene = jax.lax.broadcasted_iota(jnp.int32, (R, 1), 0) // N
    cw = jnp.where(mask, corr, -1e30)
    hb = jnp.zeros((R, R), jnp.float32)
    hc = jnp.zeros((R, N), jnp.float32)
    for _ in range(5):
        rmax = jnp.max(cw, axis=1, keepdims=True)
        eq = cw == rmax
        fidx = jnp.min(jnp.where(eq, colidx, R), axis=1, keepdims=True)
        fm = colidx == fidx
        hb = hb + fm.astype(jnp.float32)
        hc = hc + (nidx == (fidx - rowscene * N)).astype(jnp.float32)
        cw = jnp.where(fm, -1e30, cw)

    edge1 = jnp.dot(hb, ftraj) * (1.0 / 5.0)
    e1 = _relu(jnp.dot(edge1, wh1a_ref[:]) + bh1a_ref[:])
    e1 = _relu(jnp.dot(e1, wh1b_ref[:]) + bh1b_ref[:])
    ones_lane = jnp.ones((R, 128), jnp.float32)
    degf = jax.lax.dot_general(hb, ones_lane, (((0,), (0,)), ((), ())))
    deg = jnp.max(degf, axis=1, keepdims=True)
    node1 = jax.lax.dot_general(hb, e1, (((0,), (0,)), ((), ()))) \
        / jnp.maximum(deg, 1.0)

    e2 = _relu(jnp.dot(mean_rows, wh2a_ref[:]) + bh2a_ref[:])
    h2 = _relu(jnp.dot(e2, wh2b_ref[:]) + bh2b_ref[:])

    h = _relu(jnp.dot(pf_ref[:], wout_ref[0:4 * D, :])
              + jnp.dot(ftraj, wout_ref[4 * D:5 * D, :])
              + jnp.dot(inter, wout_ref[5 * D:6 * D, :])
              + jnp.dot(node1, wout_ref[6 * D:7 * D, :])
              + jnp.dot(h2, wout_ref[7 * D:8 * D, :])
              + bout_ref[:])
    qz_ref[:] = jnp.dot(h, wqz_ref[:]) + bqz_ref[:]

    h_out_ref[:, 0:N, :] = hc.reshape(BT, N, N)
    h_out_ref[:, N:2 * N, :] = jnp.ones((BT, N, N), jnp.float32)


def kernel(inputs, batch_size, agent_num, past_feature, Wf, bf, Wpe, bpe,
           W2, b2, W3, b3, Wo1, bo1, Wo2, bo2, Wh1a, bh1a, Wh1b, bh1b,
           Wh2a, bh2a, Wh2b, bh2b, Wout, bout, Wqz, bqz):
    del batch_size, agent_num

    bff = jnp.bfloat16
    pe = _pe_const()
    peb = jnp.matmul(pe.astype(bff), Wpe[D:].astype(bff),
                     preferred_element_type=jnp.float32) + bpe
    catb = jnp.matmul(_cat3_const(),
                      W3[D:].astype(bff).astype(jnp.float32)) + b3
    catb = jnp.tile(catb, (BT, 1))

    x = inputs.reshape(GRID, R, T, IN_DIM).transpose(0, 2, 1, 3)
    x = x.reshape(GRID * T * R, IN_DIM)
    wo1a, wo1b = Wo1[:D], Wo1[D:]

    def _const_spec(shape):
        return pl.BlockSpec(shape, lambda i: tuple(0 for _ in shape))

    out = pl.pallas_call(
        _block_kernel,
        grid=(GRID,),
        in_specs=[
            pl.BlockSpec((T * R, IN_DIM), lambda i: (i, 0)),
            pl.BlockSpec((R, 4 * D), lambda i: (i, 0)),
            _const_spec((IN_DIM, D)),
            _const_spec((1, D)),
            _const_spec((D, D)),
            _const_spec((T, D)),
            _const_spec((T * D, D)),
            _const_spec((1, D)),
            _const_spec((D, D)),
            _const_spec((R, D)),
            _const_spec((D, 64)),
            _const_spec((D, 64)),
            _const_spec((1, 64)),
            _const_spec((64, D)),
            _const_spec((1, D)),
            _const_spec((D, 64)),
            _const_spec((1, 64)),
            _const_spec((64, D)),
            _const_spec((1, D)),
            _const_spec((D, 64)),
            _const_spec((1, 64)),
            _const_spec((64, D)),
            _const_spec((1, D)),
            _const_spec((8 * D, 128)),
            _const_spec((1, 128)),
            _const_spec((128, 2 * ZDIM)),
            _const_spec((1, 2 * ZDIM)),
        ],
        out_specs=[
            pl.BlockSpec((R, 2 * ZDIM), lambda i: (i, 0)),
            pl.BlockSpec((BT, 2 * N, N), lambda i: (i, 0, 0)),
        ],
        out_shape=[
            jax.ShapeDtypeStruct((B * N, 2 * ZDIM), jnp.float32),
            jax.ShapeDtypeStruct((B, 2 * N, N), jnp.float32),
        ],
        compiler_params=pltpu.CompilerParams(
            dimension_semantics=("parallel",)),
    )(x, past_feature, Wf.astype(bff), bf[None], Wpe[:D].astype(bff), peb,
      W2.astype(bff), b2[None], W3[:D].astype(bff), catb,
      wo1a, wo1b, bo1[None], Wo2, bo2[None],
      Wh1a, bh1a[None], Wh1b, bh1b[None],
      Wh2a, bh2a[None], Wh2b, bh2b[None],
      Wout, bout[None], Wqz, bqz[None])
    return out[0], out[1]


# speedup vs baseline: 1.0928x; 1.0928x over previous
import numpy as np
import jax
import jax.numpy as jnp
from jax.experimental import pallas as pl
from jax.experimental.pallas import tpu as pltpu

B = 1024
N = 11
T = 10
IN_DIM = 4
D = 256
ZDIM = 32
BT = 32
R = BT * N
GRID = B // BT

_HP = jax.lax.Precision.HIGHEST


def _pe_const():
    position = np.arange(200, dtype=np.float32)[:, None]
    div_term = np.exp(np.arange(0, D, 2, dtype=np.float32) * (-np.log(10000.0) / D))
    pe = np.zeros((200, D), dtype=np.float32)
    pe[:, 0::2] = np.sin(position * div_term)
    pe[:, 1::2] = np.cos(position * div_term)
    return jnp.asarray(pe[:T])


def _cat3_const():
    cat3 = np.zeros((N, 3), dtype=np.float32)
    cat3[0:5, 0] = 1.0
    cat3[5:10, 1] = 1.0
    cat3[10, 2] = 1.0
    return jnp.asarray(cat3)


def _relu(x):
    return jnp.maximum(x, 0.0)


def _bfdot(a, b):
    return jax.lax.dot_general(a.astype(jnp.bfloat16), b, (((1,), (0,)), ((), ())),
                               preferred_element_type=jnp.float32)


def _block_kernel(x_ref, pf_ref,
                  wf_ref, bf_ref, wpet_ref, peb_ref, w2_ref, b2_ref,
                  w3a_ref, catb_ref,
                  wo1a_ref, wo1b_ref, bo1_ref, wo2_ref, bo2_ref,
                  wh1a_ref, bh1a_ref, wh1b_ref, bh1b_ref,
                  wh2a_ref, bh2a_ref, wh2b_ref, bh2b_ref,
                  wout_ref, bout_ref, wqz_ref, bqz_ref,
                  qz_ref, h_out_ref):
    x = x_ref[:]
    tf_all = _bfdot(x, wf_ref[:]) + bf_ref[:]
    parts = []
    for t in range(T):
        tfp = _bfdot(tf_all[:, t * D:(t + 1) * D], wpet_ref[:]) + peb_ref[t:t + 1, :]
        parts.append(tfp)
    f_in = jnp.concatenate(parts, axis=1)
    f0 = _bfdot(f_in, w2_ref[:]) + b2_ref[:]
    ftraj = _bfdot(f0, w3a_ref[:]) + catb_ref[:]

    nrm = jnp.sqrt(jnp.sum(ftraj * ftraj, axis=1, keepdims=True))
    q = (ftraj / jnp.maximum(nrm, 1e-12)).astype(jnp.bfloat16)
    corr = jax.lax.dot_general(q, q, (((1,), (1,)), ((), ())),
                               preferred_element_type=jnp.float32)

    rs = jax.lax.broadcasted_iota(jnp.int32, (R, R), 0) // N
    cs = jax.lax.broadcasted_iota(jnp.int32, (R, R), 1) // N
    mask = rs == cs
    maskf = mask.astype(jnp.float32)

    mean_rows = jnp.dot(maskf, ftraj) * (1.0 / N)

    m = _relu(jnp.dot(ftraj, wo1a_ref[:])
              + jnp.dot(mean_rows, wo1b_ref[:]) + bo1_ref[:])
    inter = _relu(jnp.dot(m, wo2_ref[:]) + bo2_ref[:])

    colidx = jax.lax.broadcasted_iota(jnp.int32, (R, R), 1)
    nidx = jax.lax.broadcasted_iota(jnp.int32, (R, N), 1)
    rowscene = jax.lax.broadcasted_iota(jnp.int32, (R, 1), 0) // N
    cw = jnp.where(mask, corr, -1e30)
    hb = jnp.zeros((R, R), jnp.float32)
    hc = jnp.zeros((R, N), jnp.float32)
    for _ in range(5):
        rmax = jnp.max(cw, axis=1, keepdims=True)
        eq = cw == rmax
        fidx = jnp.min(jnp.where(eq, colidx, R), axis=1, keepdims=True)
        fm = colidx == fidx
        hb = hb + fm.astype(jnp.float32)
        hc = hc + (nidx == (fidx - rowscene * N)).astype(jnp.float32)
        cw = jnp.where(fm, -1e30, cw)

    edge1 = jnp.dot(hb, ftraj) * (1.0 / 5.0)
    e1 = _relu(jnp.dot(edge1, wh1a_ref[:]) + bh1a_ref[:])
    e1 = _relu(jnp.dot(e1, wh1b_ref[:]) + bh1b_ref[:])
    ones_lane = jnp.ones((R, 128), jnp.float32)
    degf = jax.lax.dot_general(hb, ones_lane, (((0,), (0,)), ((), ())))
    deg = jnp.max(degf, axis=1, keepdims=True)
    node1 = jax.lax.dot_general(hb, e1, (((0,), (0,)), ((), ()))) \
        / jnp.maximum(deg, 1.0)

    e2 = _relu(jnp.dot(mean_rows, wh2a_ref[:]) + bh2a_ref[:])
    h2 = _relu(jnp.dot(e2, wh2b_ref[:]) + bh2b_ref[:])

    h = _relu(jnp.dot(pf_ref[:], wout_ref[0:4 * D, :])
              + jnp.dot(ftraj, wout_ref[4 * D:5 * D, :])
              + jnp.dot(inter, wout_ref[5 * D:6 * D, :])
              + jnp.dot(node1, wout_ref[6 * D:7 * D, :])
              + jnp.dot(h2, wout_ref[7 * D:8 * D, :])
              + bout_ref[:])
    qz_ref[:] = jnp.dot(h, wqz_ref[:]) + bqz_ref[:]

    h_out_ref[:, 0:N, :] = hc.reshape(BT, N, N)
    h_out_ref[:, N:2 * N, :] = jnp.ones((BT, N, N), jnp.float32)


def kernel(inputs, batch_size, agent_num, past_feature, Wf, bf, Wpe, bpe,
           W2, b2, W3, b3, Wo1, bo1, Wo2, bo2, Wh1a, bh1a, Wh1b, bh1b,
           Wh2a, bh2a, Wh2b, bh2b, Wout, bout, Wqz, bqz):
    del batch_size, agent_num

    bff = jnp.bfloat16
    wfblk = jnp.kron(jnp.eye(T, dtype=jnp.float32), Wf)
    bft = jnp.tile(bf, (T,))[None]
    pe = _pe_const()
    peb = jnp.matmul(pe.astype(bff), Wpe[D:].astype(bff),
                     preferred_element_type=jnp.float32) + bpe
    catb = jnp.matmul(_cat3_const(),
                      W3[D:].astype(bff).astype(jnp.float32)) + b3
    catb = jnp.tile(catb, (BT, 1))

    x = inputs.reshape(B * N, T * IN_DIM)
    wo1a, wo1b = Wo1[:D], Wo1[D:]

    def _const_spec(shape):
        return pl.BlockSpec(shape, lambda i: tuple(0 for _ in shape))

    out = pl.pallas_call(
        _block_kernel,
        grid=(GRID,),
        in_specs=[
            pl.BlockSpec((R, T * IN_DIM), lambda i: (i, 0)),
            pl.BlockSpec((R, 4 * D), lambda i: (i, 0)),
            _const_spec((T * IN_DIM, T * D)),
            _const_spec((1, T * D)),
            _const_spec((D, D)),
            _const_spec((T, D)),
            _const_spec((T * D, D)),
            _const_spec((1, D)),
            _const_spec((D, D)),
            _const_spec((R, D)),
            _const_spec((D, 64)),
            _const_spec((D, 64)),
            _const_spec((1, 64)),
            _const_spec((64, D)),
            _const_spec((1, D)),
            _const_spec((D, 64)),
            _const_spec((1, 64)),
            _const_spec((64, D)),
            _const_spec((1, D)),
            _const_spec((D, 64)),
            _const_spec((1, 64)),
            _const_spec((64, D)),
            _const_spec((1, D)),
            _const_spec((8 * D, 128)),
            _const_spec((1, 128)),
            _const_spec((128, 2 * ZDIM)),
            _const_spec((1, 2 * ZDIM)),
        ],
        out_specs=[
            pl.BlockSpec((R, 2 * ZDIM), lambda i: (i, 0)),
            pl.BlockSpec((BT, 2 * N, N), lambda i: (i, 0, 0)),
        ],
        out_shape=[
            jax.ShapeDtypeStruct((B * N, 2 * ZDIM), jnp.float32),
            jax.ShapeDtypeStruct((B, 2 * N, N), jnp.float32),
        ],
        compiler_params=pltpu.CompilerParams(
            dimension_semantics=("parallel",)),
    )(x, past_feature, wfblk.astype(bff), bft, Wpe[:D].astype(bff), peb,
      W2.astype(bff), b2[None], W3[:D].astype(bff), catb,
      wo1a, wo1b, bo1[None], Wo2, bo2[None],
      Wh1a, bh1a[None], Wh1b, bh1b[None],
      Wh2a, bh2a[None], Wh2b, bh2b[None],
      Wout, bout[None], Wqz, bqz[None])
    return out[0], out[1]


# speedup vs baseline: 1.0978x; 1.0046x over previous
import numpy as np
import jax
import jax.numpy as jnp
from jax.experimental import pallas as pl
from jax.experimental.pallas import tpu as pltpu

B = 1024
N = 11
T = 10
IN_DIM = 4
D = 256
ZDIM = 32
BT = 32
R = BT * N
GRID = B // BT


def _pe_const():
    position = np.arange(200, dtype=np.float32)[:, None]
    div_term = np.exp(np.arange(0, D, 2, dtype=np.float32) * (-np.log(10000.0) / D))
    pe = np.zeros((200, D), dtype=np.float32)
    pe[:, 0::2] = np.sin(position * div_term)
    pe[:, 1::2] = np.cos(position * div_term)
    return jnp.asarray(pe[:T])


def _cat3_const():
    cat3 = np.zeros((N, 3), dtype=np.float32)
    cat3[0:5, 0] = 1.0
    cat3[5:10, 1] = 1.0
    cat3[10, 2] = 1.0
    return jnp.asarray(cat3)


def _relu(x):
    return jnp.maximum(x, 0.0)


def _bfdot(a, b):
    return jax.lax.dot_general(a.astype(jnp.bfloat16), b, (((1,), (0,)), ((), ())),
                               preferred_element_type=jnp.float32)


def _block_kernel(x_ref, pf_ref,
                  wf_ref, bf_ref, wpet_ref, peb_ref, w2_ref, b2_ref,
                  w3a_ref, catb_ref,
                  wo1a_ref, wo1b_ref, bo1_ref, wo2_ref, bo2_ref,
                  wh1a_ref, bh1a_ref, wh1b_ref, bh1b_ref,
                  wh2a_ref, bh2a_ref, wh2b_ref, bh2b_ref,
                  wout_ref, bout_ref, wqz_ref, bqz_ref,
                  qz_ref, h_out_ref):
    x = x_ref[:]
    tf_all = _bfdot(x, wf_ref[:]) + bf_ref[:]
    parts = []
    for t in range(T):
        tfp = _bfdot(tf_all[:, t * D:(t + 1) * D], wpet_ref[:]) + peb_ref[t:t + 1, :]
        parts.append(tfp)
    f_in = jnp.concatenate(parts, axis=1)
    f0 = _bfdot(f_in, w2_ref[:]) + b2_ref[:]
    ftraj = _bfdot(f0, w3a_ref[:]) + catb_ref[:]

    nrm = jnp.sqrt(jnp.sum(ftraj * ftraj, axis=1, keepdims=True))
    q = (ftraj / jnp.maximum(nrm, 1e-12)).astype(jnp.bfloat16)
    corr = jax.lax.dot_general(q, q, (((1,), (1,)), ((), ())),
                               preferred_element_type=jnp.float32)

    rs = jax.lax.broadcasted_iota(jnp.int32, (R, R), 0) // N
    cs = jax.lax.broadcasted_iota(jnp.int32, (R, R), 1) // N
    mask = rs == cs
    maskf = mask.astype(jnp.float32)

    mean_rows = jnp.dot(maskf, ftraj) * (1.0 / N)

    m = _relu(jnp.dot(ftraj, wo1a_ref[:])
              + jnp.dot(mean_rows, wo1b_ref[:]) + bo1_ref[:])
    inter = _relu(jnp.dot(m, wo2_ref[:]) + bo2_ref[:])

    colidx = jax.lax.broadcasted_iota(jnp.int32, (R, R), 1)
    nidx = jax.lax.broadcasted_iota(jnp.int32, (R, N), 1)
    rowscene = jax.lax.broadcasted_iota(jnp.int32, (R, 1), 0) // N
    cw = jnp.where(mask, corr, -1e30)
    hb = jnp.zeros((R, R), jnp.float32)
    hc = jnp.zeros((R, N), jnp.float32)
    for _ in range(5):
        rmax = jnp.max(cw, axis=1, keepdims=True)
        eq = cw == rmax
        fidx = jnp.min(jnp.where(eq, colidx, R), axis=1, keepdims=True)
        fm = colidx == fidx
        hb = hb + fm.astype(jnp.float32)
        hc = hc + (nidx == (fidx - rowscene * N)).astype(jnp.float32)
        cw = jnp.where(fm, -1e30, cw)

    edge1 = jnp.dot(hb, ftraj) * (1.0 / 5.0)
    e1 = _relu(jnp.dot(edge1, wh1a_ref[:]) + bh1a_ref[:])
    e1 = _relu(jnp.dot(e1, wh1b_ref[:]) + bh1b_ref[:])
    ones_lane = jnp.ones((R, 128), jnp.float32)
    degf = jax.lax.dot_general(hb, ones_lane, (((0,), (0,)), ((), ())))
    deg = jnp.max(degf, axis=1, keepdims=True)
    node1 = jax.lax.dot_general(hb, e1, (((0,), (0,)), ((), ()))) \
        / jnp.maximum(deg, 1.0)

    e2 = _relu(jnp.dot(mean_rows, wh2a_ref[:]) + bh2a_ref[:])
    h2 = _relu(jnp.dot(e2, wh2b_ref[:]) + bh2b_ref[:])

    h = _relu(jnp.dot(pf_ref[:], wout_ref[0:4 * D, :])
              + jnp.dot(ftraj, wout_ref[4 * D:5 * D, :])
              + jnp.dot(inter, wout_ref[5 * D:6 * D, :])
              + jnp.dot(node1, wout_ref[6 * D:7 * D, :])
              + jnp.dot(h2, wout_ref[7 * D:8 * D, :])
              + bout_ref[:])
    qz_ref[:] = jnp.dot(h, wqz_ref[:]) + bqz_ref[:]

    h_out_ref[:, 0:N, :] = hc.reshape(BT, N, N)
    h_out_ref[:, N:2 * N, :] = jnp.ones((BT, N, N), jnp.float32)


def kernel(inputs, batch_size, agent_num, past_feature, Wf, bf, Wpe, bpe,
           W2, b2, W3, b3, Wo1, bo1, Wo2, bo2, Wh1a, bh1a, Wh1b, bh1b,
           Wh2a, bh2a, Wh2b, bh2b, Wout, bout, Wqz, bqz):
    del batch_size, agent_num

    bff = jnp.bfloat16
    wfblk = jnp.kron(jnp.eye(T, dtype=jnp.float32), Wf)
    bft = jnp.tile(bf, (T,))[None]
    pe = _pe_const()
    peb = jnp.matmul(pe.astype(bff), Wpe[D:].astype(bff),
                     preferred_element_type=jnp.float32) + bpe
    catb = jnp.matmul(_cat3_const(),
                      W3[D:].astype(bff).astype(jnp.float32)) + b3
    catb = jnp.tile(catb, (BT, 1))

    x = inputs.reshape(B * N, T * IN_DIM)
    wo1a, wo1b = Wo1[:D], Wo1[D:]

    def _const_spec(shape):
        return pl.BlockSpec(shape, lambda i: tuple(0 for _ in shape))

    out = pl.pallas_call(
        _block_kernel,
        grid=(GRID,),
        in_specs=[
            pl.BlockSpec((R, T * IN_DIM), lambda i: (i, 0)),
            pl.BlockSpec((R, 4 * D), lambda i: (i, 0)),
            _const_spec((T * IN_DIM, T * D)),
            _const_spec((1, T * D)),
            _const_spec((D, D)),
            _const_spec((T, D)),
            _const_spec((T * D, D)),
            _const_spec((1, D)),
            _const_spec((D, D)),
            _const_spec((R, D)),
            _const_spec((D, 64)),
            _const_spec((D, 64)),
            _const_spec((1, 64)),
            _const_spec((64, D)),
            _const_spec((1, D)),
            _const_spec((D, 64)),
            _const_spec((1, 64)),
            _const_spec((64, D)),
            _const_spec((1, D)),
            _const_spec((D, 64)),
            _const_spec((1, 64)),
            _const_spec((64, D)),
            _const_spec((1, D)),
            _const_spec((8 * D, 128)),
            _const_spec((1, 128)),
            _const_spec((128, 2 * ZDIM)),
            _const_spec((1, 2 * ZDIM)),
        ],
        out_specs=[
            pl.BlockSpec((R, 2 * ZDIM), lambda i: (i, 0)),
            pl.BlockSpec((BT, 2 * N, N), lambda i: (i, 0, 0)),
        ],
        out_shape=[
            jax.ShapeDtypeStruct((B * N, 2 * ZDIM), jnp.float32),
            jax.ShapeDtypeStruct((B, 2 * N, N), jnp.float32),
        ],
        compiler_params=pltpu.CompilerParams(
            dimension_semantics=("parallel",)),
    )(x, past_feature, wfblk.astype(bff), bft, Wpe[:D].astype(bff), peb,
      W2.astype(bff), b2[None], W3[:D].astype(bff), catb,
      wo1a, wo1b, bo1[None], Wo2, bo2[None],
      Wh1a, bh1a[None], Wh1b, bh1b[None],
      Wh2a, bh2a[None], Wh2b, bh2b[None],
      Wout, bout[None], Wqz, bqz[None])
    return out[0], out[1]
